# baseline (speedup 1.0000x reference)
import jax
import jax.numpy as jnp
from jax import lax
from jax.experimental import pallas as pl
from jax.experimental.pallas import tpu as pltpu
from jax.experimental.pallas import tpu_sc as plsc

EMBED = 32
NC = 2
NS = 16
NW = NC * NS
BT = 128
HIST = 50


def _build(B, V):
    n_bt = B // HIST // BT
    bt_per_w = n_bt // NW
    bw = bt_per_w * BT

    def body(table_hbm, idxT_hbm, x_hbm, idx_v, rows_v, tb_v,
             sg0, sg1, ss0, ss1):
        wid = lax.axis_index("s") * NC + lax.axis_index("c")
        bt0 = wid * bt_per_w

        pltpu.sync_copy(idxT_hbm.at[:, pl.ds(bt0 * BT, bw)], idx_v)

        def gather_start(h, buf, sem):
            pltpu.async_copy(table_hbm.at[idx_v.at[h]], rows_v.at[buf],
                             sem)

        def gather_wait(h, buf, sem):
            pltpu.make_async_copy(table_hbm.at[idx_v.at[h]],
                                  rows_v.at[buf], sem).wait()

        e0 = lax.iota(jnp.int32, 16)

        def transpose(buf, tbuf):
            def l_body(l0, carry):
                for j in range(16):
                    l = l0 * 16 + j
                    btl = l // BT
                    bl = lax.rem(l, BT)
                    bv = jnp.zeros((16,), jnp.int32) + btl
                    lv = jnp.zeros((16,), jnp.int32) + bl
                    for g2 in range(2):
                        ev = e0 + 16 * g2
                        v = rows_v[buf, l, pl.ds(g2 * 16, 16)]
                        plsc.store_scatter(
                            tb_v.at[tbuf],
                            [ev // 8, bv, lax.rem(ev, 8), lv], v)
                return carry
            lax.fori_loop(0, bw // 16, l_body, 0)

        def out_start(h, tbuf, sem):
            pltpu.async_copy(tb_v.at[tbuf, :, :, :, pl.ds(0, BT)],
                             x_hbm.at[h, :, pl.ds(bt0, bt_per_w)], sem)

        def out_wait(h, tbuf, sem):
            pltpu.make_async_copy(tb_v.at[tbuf, :, :, :, pl.ds(0, BT)],
                                  x_hbm.at[h, :, pl.ds(bt0, bt_per_w)],
                                  sem).wait()

        sgs = (sg0, sg1)
        sss = (ss0, ss1)
        gather_start(0, 0, sg0)
        gather_start(1, 1, sg1)

        def step(h, buf, kk):
            gather_wait(h, buf, sgs[buf])

            @pl.when(kk > 0)
            def _():
                out_wait(h - 2, buf, sss[buf])
            transpose(buf, buf)
            out_start(h, buf, sss[buf])

            @pl.when(h + 2 < HIST)
            def _():
                gather_start(h + 2, buf, sgs[buf])

        def outer(kk, carry):
            step(2 * kk, 0, kk)
            step(2 * kk + 1, 1, kk)
            return carry

        lax.fori_loop(0, HIST // 2, outer, 0)
        out_wait(HIST - 2, 0, ss0)
        out_wait(HIST - 1, 1, ss1)

    mesh = plsc.VectorSubcoreMesh(
        core_axis_name="c", subcore_axis_name="s", num_cores=NC,
        num_subcores=NS,
    )
    return pl.kernel(
        body,
        out_type=jax.ShapeDtypeStruct(
            (HIST, EMBED // 8, n_bt, 8, BT), jnp.float32),
        mesh=mesh,
        compiler_params=pltpu.CompilerParams(
            use_tc_tiling_on_sc=False, needs_layout_passes=False
        ),
        scratch_types=[
            pltpu.VMEM((HIST, bw), jnp.int32),
            pltpu.VMEM((2, bw, EMBED), jnp.float32),
            pltpu.VMEM((2, EMBED // 8, bt_per_w, 8, BT + 1), jnp.float32),
            pltpu.SemaphoreType.DMA,
            pltpu.SemaphoreType.DMA,
            pltpu.SemaphoreType.DMA,
            pltpu.SemaphoreType.DMA,
        ],
    )


def kernel(inputs, table):
    B = inputs.size
    idxT = inputs.T.astype(jnp.int32)
    x = _build(B, table.shape[0])(table, idxT)
    out = x.transpose(2, 4, 0, 1, 3).reshape(B // HIST, HIST, EMBED)
    return out
